# baseline (speedup 1.0000x reference)
import functools

import jax
import jax.numpy as jnp
from jax import lax
from jax.experimental import pallas as pl
from jax.experimental.pallas import tpu as pltpu
from jax.experimental.pallas import tpu_sc as plsc

NW = 32
L = 16
NB = 2


@functools.cache
def _make(batch, seq, vocab, dim):
    cols = batch // NW
    mesh = plsc.VectorSubcoreMesh(core_axis_name="c", subcore_axis_name="s")

    @functools.partial(
        pl.kernel,
        mesh=mesh,
        out_type=jax.ShapeDtypeStruct((seq, dim, batch), jnp.float32),
        scratch_types=[
            pltpu.VMEM((seq, cols), jnp.int32),
            pltpu.VMEM((NB, cols), jnp.int32),
            pltpu.VMEM((NB, cols, 2 * dim), jnp.float32),
            pltpu.VMEM((NB, dim, cols), jnp.float32),
            pltpu.SemaphoreType.DMA((NB,)),
            pltpu.SemaphoreType.DMA((NB,)),
        ],
        compiler_params=pltpu.CompilerParams(needs_layout_passes=False),
    )
    def k(idx_hbm, table_hbm, out_hbm, idx_v, idx2_v, g_v, gt_v, gsem, osem):
        wid = lax.axis_index("s") * 2 + lax.axis_index("c")
        cb = wid * cols
        pltpu.sync_copy(idx_hbm.at[:, pl.ds(cb, cols)], idx_v)

        def prep_and_gather(s, b):
            for j in range(cols // L):
                v = idx_v[s, pl.ds(j * L, L)]
                idx2_v[b, pl.ds(j * L, L)] = lax.shift_right_logical(v, 1)
            pltpu.async_copy(table_hbm.at[idx2_v.at[b]], g_v.at[b], gsem.at[b])

        def gwait(b):
            pltpu.make_async_copy(
                table_hbm.at[idx2_v.at[b]], g_v.at[b], gsem.at[b]
            ).wait()

        def write(s, b):
            return pltpu.async_copy(
                gt_v.at[b], out_hbm.at[s, :, pl.ds(cb, cols)], osem.at[b]
            )

        def owait(s, b):
            pltpu.make_async_copy(
                gt_v.at[b], out_hbm.at[s, :, pl.ds(cb, cols)], osem.at[b]
            ).wait()

        def transpose(s, b):
            for j in range(cols // L):
                rowv = lax.broadcasted_iota(jnp.int32, (L,), 0) + (j * L)
                half = lax.shift_left(idx_v[s, pl.ds(j * L, L)] & 1, 6)
                for e in range(dim):
                    vals = plsc.load_gather(g_v.at[b], [rowv, half + e])
                    gt_v[b, e, pl.ds(j * L, L)] = vals

        for b in range(NB):
            prep_and_gather(b, b)

        def body(g, carry):
            for b in range(NB):
                s = g * NB + b
                gwait(b)

                @pl.when(g > 0)
                def _():
                    owait(s - NB, b)

                transpose(s, b)
                write(s, b)

                @pl.when(g < seq // NB - 1)
                def _():
                    prep_and_gather(s + NB, b)

            return carry

        lax.fori_loop(0, seq // NB, body, 0)
        for b in range(NB):
            owait(seq - NB + b, b)

    return k


def kernel(inputs, embedding):
    batch, seq = inputs.shape
    vocab, dim = embedding.shape
    idx_t = inputs.T
    table2 = embedding.reshape(vocab // 2, 2 * dim)
    out_t = _make(batch, seq, vocab, dim)(idx_t, table2)
    return out_t.transpose(2, 0, 1)


# speedup vs baseline: 1.9304x; 1.9304x over previous
import functools

import jax
import jax.numpy as jnp
from jax import lax
from jax.experimental import pallas as pl
from jax.experimental.pallas import tpu as pltpu
from jax.experimental.pallas import tpu_sc as plsc

NW = 32
CHUNK = 128
NBUF = 4


@functools.cache
def _make(total, vocab, dim):
    per_w = total // NW
    n_chunks = per_w // CHUNK
    n_groups = n_chunks // NBUF
    mesh = plsc.VectorSubcoreMesh(core_axis_name="c", subcore_axis_name="s")

    @functools.partial(
        pl.kernel,
        mesh=mesh,
        out_type=jax.ShapeDtypeStruct((total, 2 * dim), jnp.float32),
        scratch_types=[
            pltpu.VMEM((n_chunks, CHUNK), jnp.int32),
            pltpu.VMEM((NBUF, CHUNK, 2 * dim), jnp.float32),
            pltpu.SemaphoreType.DMA((NBUF,)),
            pltpu.SemaphoreType.DMA((NBUF,)),
        ],
        compiler_params=pltpu.CompilerParams(needs_layout_passes=False),
    )
    def k(idx_hbm, table_hbm, out_hbm, idx_v, rows_v, gsem, osem):
        wid = lax.axis_index("s") * 2 + lax.axis_index("c")
        base = wid * per_w
        pltpu.sync_copy(idx_hbm.at[wid], idx_v)

        def gather(i, b):
            return pltpu.async_copy(
                table_hbm.at[idx_v.at[i]], rows_v.at[b], gsem.at[b]
            )

        def writeback(i, b):
            return pltpu.async_copy(
                rows_v.at[b], out_hbm.at[pl.ds(base + i * CHUNK, CHUNK)], osem.at[b]
            )

        for b in range(NBUF):
            gather(b, b)

        def body(g, carry):
            for b in range(NBUF):
                i = g * NBUF + b
                pltpu.make_async_copy(
                    table_hbm.at[idx_v.at[i]], rows_v.at[b], gsem.at[b]
                ).wait()
                writeback(i, b).wait()

                @pl.when(g < n_groups - 1)
                def _():
                    gather(i + NBUF, b)

            return carry

        lax.fori_loop(0, n_groups, body, 0)

    return k


def kernel(inputs, embedding):
    batch, seq = inputs.shape
    vocab, dim = embedding.shape
    total = batch * seq
    table128 = jnp.pad(embedding, ((0, 0), (0, 128 - dim)))
    idx3 = inputs.reshape(NW, total // NW // CHUNK, CHUNK)
    out = _make(total, vocab, dim)(idx3, table128)
    return out[:, :dim].reshape(batch, seq, dim)


# speedup vs baseline: 1.9306x; 1.0001x over previous
import functools

import jax
import jax.numpy as jnp
from jax import lax
from jax.experimental import pallas as pl
from jax.experimental.pallas import tpu as pltpu
from jax.experimental.pallas import tpu_sc as plsc

NW = 32
CHUNK = 128
NBUF = 5


@functools.cache
def _make(total, vocab, dim):
    per_w = total // NW
    n_chunks = per_w // CHUNK
    n_groups = n_chunks // NBUF
    mesh = plsc.VectorSubcoreMesh(core_axis_name="c", subcore_axis_name="s")

    @functools.partial(
        pl.kernel,
        mesh=mesh,
        out_type=jax.ShapeDtypeStruct((total, 2 * dim), jnp.float32),
        scratch_types=[
            pltpu.VMEM((n_chunks, CHUNK), jnp.int32),
            pltpu.VMEM((NBUF, CHUNK, 2 * dim), jnp.float32),
            pltpu.SemaphoreType.DMA((NBUF,)),
            pltpu.SemaphoreType.DMA((NBUF,)),
        ],
        compiler_params=pltpu.CompilerParams(needs_layout_passes=False),
    )
    def k(idx_hbm, table_hbm, out_hbm, idx_v, rows_v, gsem, osem):
        wid = lax.axis_index("s") * 2 + lax.axis_index("c")
        base = wid * per_w
        pltpu.sync_copy(idx_hbm.at[wid], idx_v)

        def gather(i, b):
            return pltpu.async_copy(
                table_hbm.at[idx_v.at[i]], rows_v.at[b], gsem.at[b]
            )

        def writeback(i, b):
            return pltpu.async_copy(
                rows_v.at[b], out_hbm.at[pl.ds(base + i * CHUNK, CHUNK)], osem.at[b]
            )

        for b in range(NBUF):
            gather(b, b)

        def body(g, carry):
            for b in range(NBUF):
                i = g * NBUF + b
                pltpu.make_async_copy(
                    table_hbm.at[idx_v.at[i]], rows_v.at[b], gsem.at[b]
                ).wait()
                writeback(i, b).wait()

                @pl.when(g < n_groups - 1)
                def _():
                    gather(i + NBUF, b)

            return carry

        lax.fori_loop(0, n_groups, body, 0)

    return k


def kernel(inputs, embedding):
    batch, seq = inputs.shape
    vocab, dim = embedding.shape
    total = batch * seq
    table128 = jnp.pad(embedding, ((0, 0), (0, 128 - dim)))
    idx3 = inputs.reshape(NW, total // NW // CHUNK, CHUNK)
    out = _make(total, vocab, dim)(idx3, table128)
    return out[:, :dim].reshape(batch, seq, dim)
